# Initial kernel scaffold; baseline (speedup 1.0000x reference)
import jax
import jax.numpy as jnp
from jax.experimental import pallas as pl


def kernel(x, index, weights, gW1, gb1, gW2, gb2, mW1, mb1, mW2, mb2, p):
    raise NotImplementedError("write your pallas kernel here")


# speedup vs baseline: 3.2496x; 3.2496x over previous
import dataclasses
import functools

import jax
import jax.numpy as jnp
from jax import lax
from jax.experimental import pallas as pl
from jax.experimental.pallas import tpu as pltpu
from jax.experimental.pallas import tpu_sc as plsc

_S = 10000
_SP = 10240
_D = 128
_B = 1280
_NC = 2
_NT = 16
_NW = _NC * _NT
_SPT = _SP // _NW
_ACCR = _SPT + 8
_CHUNK = 80
_EPS = 1e-10

_SELU_ALPHA = 1.6732632423543772848170429916717
_SELU_SCALE = 1.0507009873554804934193349852946


def _selu(x):
    return _SELU_SCALE * jnp.where(x > 0, x, _SELU_ALPHA * (jnp.exp(x) - 1.0))


def _mlp_body(x_ref, w_ref, gW1_ref, gb1_ref, gW2_ref, gb2_ref,
              mW1_ref, mb1_ref, mW2_ref, mb2_ref, p_ref,
              out_ref, s_ref):
    x = x_ref[...]
    h = jnp.dot(x, gW1_ref[...], preferred_element_type=jnp.float32)
    h = _selu(h + gb1_ref[...])
    g = jnp.dot(h, gW2_ref[...], preferred_element_type=jnp.float32)
    g = g + gb2_ref[...]
    w = w_ref[...]
    t = p_ref[0, 0] * jnp.log(jnp.maximum(w, _EPS)) + g
    s = jnp.exp(jnp.clip(t, -80.0, 80.0))
    m = _selu(jnp.dot(x, mW1_ref[...], preferred_element_type=jnp.float32)
              + mb1_ref[...])
    m = _selu(jnp.dot(m, mW2_ref[...], preferred_element_type=jnp.float32)
              + mb2_ref[...])
    out_ref[...] = s * m
    s_ref[...] = jnp.concatenate(
        [s, jnp.zeros((s.shape[0], 15), jnp.float32)], axis=1)


def _mlp_call(x, w2d, gW1, gb1, gW2, gb2, mW1, mb1, mW2, mb2, p2d):
    n = x.shape[0]
    grid = (n // _B,)
    full = lambda shape: pl.BlockSpec(shape, lambda i: (0, 0))
    return pl.pallas_call(
        _mlp_body,
        grid=grid,
        in_specs=[
            pl.BlockSpec((_B, _D), lambda i: (i, 0)),
            pl.BlockSpec((_B, 1), lambda i: (i, 0)),
            full((_D, 64)), full((1, 64)),
            full((64, 1)), full((1, 1)),
            full((_D, _D)), full((1, _D)),
            full((_D, _D)), full((1, _D)),
            full((1, 1)),
        ],
        out_specs=[
            pl.BlockSpec((_B, _D), lambda i: (i, 0)),
            pl.BlockSpec((_B, 16), lambda i: (i, 0)),
        ],
        out_shape=[
            jax.ShapeDtypeStruct((n, _D), jnp.float32),
            jax.ShapeDtypeStruct((n, 16), jnp.float32),
        ],
    )(x, w2d, gW1, gb1, gW2, gb2, mW1, mb1, mW2, mb2, p2d)


def _extract16(vec, lane):
    m = lax.iota(jnp.int32, 16) == lane
    return jnp.max(jnp.where(m, vec, 0))


def _make_scatter_kernel(n):
    mesh = plsc.VectorSubcoreMesh(core_axis_name="c", subcore_axis_name="s")
    cp = pltpu.CompilerParams()
    if "needs_layout_passes" in pltpu.CompilerParams.__dataclass_fields__:
        cp = dataclasses.replace(cp, needs_layout_passes=False)

    @functools.partial(
        pl.kernel,
        out_type=(jax.ShapeDtypeStruct((_SP * _D,), jnp.float32),
                  jax.ShapeDtypeStruct((_SP * 16,), jnp.float32)),
        mesh=mesh,
        compiler_params=cp,
        scratch_types=[
            pltpu.VMEM((_CHUNK * _D,), jnp.float32),
            pltpu.VMEM((_CHUNK * 16,), jnp.float32),
            pltpu.VMEM((_CHUNK,), jnp.int32),
            pltpu.VMEM((_ACCR * _D,), jnp.float32),
            pltpu.VMEM((_ACCR * 16,), jnp.float32),
            pltpu.VMEM((_NW,), jnp.int32),
            pltpu.VMEM((_NW,), jnp.int32),
            pltpu.VMEM((_NW,), jnp.int32),
        ],
    )
    def scatter_kernel(big_hbm, s16_hbm, idx_hbm, znum_hbm, zden_hbm,
                       bs_hbm, be_hbm, bn_hbm, num_out, den_out,
                       rows_v, s16_v, idx_v, acc_v, accd_v,
                       bs_v, be_v, bn_v):
        c = lax.axis_index("c")
        t = lax.axis_index("s")
        w = t * _NC + c
        lo = w * _SPT

        pltpu.sync_copy(bs_hbm, bs_v)
        pltpu.sync_copy(be_hbm, be_v)
        pltpu.sync_copy(bn_hbm, bn_v)
        g = w // 16
        lane = w - g * 16
        gds = pl.multiple_of(g * 16, 16)
        astart = pl.multiple_of(_extract16(bs_v[pl.ds(gds, 16)], lane), 8)
        aend = _extract16(be_v[pl.ds(gds, 16)], lane)
        nch = _extract16(bn_v[pl.ds(gds, 16)], lane)

        pltpu.sync_copy(znum_hbm, acc_v)
        pltpu.sync_copy(zden_hbm, accd_v)

        def chunk_body(k, carry):
            win = astart + k * _CHUNK
            off = pl.multiple_of(jnp.minimum(win, n - _CHUNK), 8)
            pltpu.sync_copy(
                big_hbm.at[pl.ds(pl.multiple_of(off * _D, 8), _CHUNK * _D)],
                rows_v)
            pltpu.sync_copy(
                s16_hbm.at[pl.ds(pl.multiple_of(off * 16, 8), _CHUNK * 16)],
                s16_v)
            pltpu.sync_copy(idx_hbm.at[pl.ds(off, _CHUNK)], idx_v)
            for g2 in range(_CHUNK // 16):
                v = idx_v[pl.ds(g2 * 16, 16)]
                rowpos = off + g2 * 16 + lax.iota(jnp.int32, 16)
                local = v - lo
                ok = ((rowpos >= win) & (rowpos < aend)
                      & (local >= 0) & (local < _SPT))
                offs = jnp.where(ok, local, _SPT) * _D
                for l in range(16):
                    o = pl.multiple_of(_extract16(offs, l), 16)
                    r = g2 * 16 + l
                    od = pl.multiple_of(
                        lax.shift_right_logical(o, 3), 16)
                    plsc.addupdate(accd_v.at[pl.ds(od, 16)],
                                   s16_v[pl.ds(r * 16, 16)])
                    for j in range(_D // 16):
                        plsc.addupdate(
                            acc_v.at[pl.ds(pl.multiple_of(o + j * 16, 16), 16)],
                            rows_v[pl.ds(r * _D + j * 16, 16)])
            return carry

        lax.fori_loop(0, nch, chunk_body, 0)

        pltpu.sync_copy(
            acc_v.at[pl.ds(0, _SPT * _D)],
            num_out.at[pl.ds(pl.multiple_of(lo * _D, 8), _SPT * _D)])
        pltpu.sync_copy(
            accd_v.at[pl.ds(0, _SPT * 16)],
            den_out.at[pl.ds(pl.multiple_of(lo * 16, 8), _SPT * 16)])

    return scatter_kernel


def _div_body(num_ref, den_ref, out_ref):
    out_ref[...] = num_ref[...] / (den_ref[:, 0:1] + _EPS)


def _div_call(num, den):
    bs = min(1000, _S)
    return pl.pallas_call(
        _div_body,
        grid=(_S // bs,),
        in_specs=[
            pl.BlockSpec((bs, _D), lambda i: (i, 0)),
            pl.BlockSpec((bs, 16), lambda i: (i, 0)),
        ],
        out_specs=pl.BlockSpec((bs, _D), lambda i: (i, 0)),
        out_shape=jax.ShapeDtypeStruct((_S, _D), jnp.float32),
    )(num, den)


def kernel(x, index, weights, gW1, gb1, gW2, gb2, mW1, mb1, mW2, mb2, p):
    n = x.shape[0]
    out_big, s16 = _mlp_call(
        x, weights.reshape(-1, 1),
        gW1, gb1.reshape(1, -1), gW2, gb2.reshape(1, 1),
        mW1, mb1.reshape(1, -1), mW2, mb2.reshape(1, -1),
        p.reshape(1, 1))

    cuts = jnp.arange(_NW + 1, dtype=jnp.int32) * _SPT
    b = jnp.searchsorted(index, cuts).astype(jnp.int32)
    astart = (b[:_NW] // 8) * 8
    aend = b[1:]
    nch = (aend - astart + (_CHUNK - 1)) // _CHUNK

    znum = jnp.zeros((_ACCR * _D,), jnp.float32)
    zden = jnp.zeros((_ACCR * 16,), jnp.float32)
    num_f, den_f = _make_scatter_kernel(n)(
        out_big.reshape(-1), s16.reshape(-1), index,
        znum, zden, astart, aend, nch)
    num = num_f.reshape(_SP, _D)[:_S]
    den = den_f.reshape(_SP, 16)[:_S]
    return _div_call(num, den)


# speedup vs baseline: 4.0373x; 1.2424x over previous
import dataclasses
import functools

import jax
import jax.numpy as jnp
from jax import lax
from jax.experimental import pallas as pl
from jax.experimental.pallas import tpu as pltpu
from jax.experimental.pallas import tpu_sc as plsc

_S = 10000
_SP = 10240
_D = 128
_B = 1280
_NC = 2
_NT = 16
_NW = _NC * _NT
_SPT = _SP // _NW
_ACCR = _SPT + 8
_CHUNK = 128
_EPS = 1e-10

_SELU_ALPHA = 1.6732632423543772848170429916717
_SELU_SCALE = 1.0507009873554804934193349852946


def _selu(x):
    return _SELU_SCALE * jnp.where(x > 0, x, _SELU_ALPHA * (jnp.exp(x) - 1.0))


def _mlp_body(x_ref, w_ref, gW1_ref, gb1_ref, gW2_ref, gb2_ref,
              mW1_ref, mb1_ref, mW2_ref, mb2_ref, p_ref,
              out_ref, s_ref):
    x = x_ref[...]
    h = jnp.dot(x, gW1_ref[...], preferred_element_type=jnp.float32)
    h = _selu(h + gb1_ref[...])
    g = jnp.dot(h, gW2_ref[...], preferred_element_type=jnp.float32)
    g = g + gb2_ref[...]
    w = w_ref[...]
    t = p_ref[0, 0] * jnp.log(jnp.maximum(w, _EPS)) + g
    s = jnp.exp(jnp.clip(t, -80.0, 80.0))
    m = _selu(jnp.dot(x, mW1_ref[...], preferred_element_type=jnp.float32)
              + mb1_ref[...])
    m = _selu(jnp.dot(m, mW2_ref[...], preferred_element_type=jnp.float32)
              + mb2_ref[...])
    out_ref[...] = s * m
    s_ref[...] = jnp.concatenate(
        [s, jnp.zeros((s.shape[0], 15), jnp.float32)], axis=1)


def _mlp_call(x, w2d, gW1, gb1, gW2, gb2, mW1, mb1, mW2, mb2, p2d):
    n = x.shape[0]
    grid = (n // _B,)
    full = lambda shape: pl.BlockSpec(shape, lambda i: (0, 0))
    return pl.pallas_call(
        _mlp_body,
        grid=grid,
        in_specs=[
            pl.BlockSpec((_B, _D), lambda i: (i, 0)),
            pl.BlockSpec((_B, 1), lambda i: (i, 0)),
            full((_D, 64)), full((1, 64)),
            full((64, 1)), full((1, 1)),
            full((_D, _D)), full((1, _D)),
            full((_D, _D)), full((1, _D)),
            full((1, 1)),
        ],
        out_specs=[
            pl.BlockSpec((_B, _D), lambda i: (i, 0)),
            pl.BlockSpec((_B, 16), lambda i: (i, 0)),
        ],
        out_shape=[
            jax.ShapeDtypeStruct((n, _D), jnp.float32),
            jax.ShapeDtypeStruct((n, 16), jnp.float32),
        ],
    )(x, w2d, gW1, gb1, gW2, gb2, mW1, mb1, mW2, mb2, p2d)


def _extract16(vec, lane):
    m = lax.iota(jnp.int32, 16) == lane
    return jnp.max(jnp.where(m, vec, 0))


def _make_scatter_kernel(n):
    mesh = plsc.VectorSubcoreMesh(core_axis_name="c", subcore_axis_name="s")
    cp = pltpu.CompilerParams()
    if "needs_layout_passes" in pltpu.CompilerParams.__dataclass_fields__:
        cp = dataclasses.replace(cp, needs_layout_passes=False)

    @functools.partial(
        pl.kernel,
        out_type=(jax.ShapeDtypeStruct((_SP * _D,), jnp.float32),
                  jax.ShapeDtypeStruct((_SP * 16,), jnp.float32)),
        mesh=mesh,
        compiler_params=cp,
        scratch_types=[
            pltpu.VMEM((_CHUNK * _D,), jnp.float32),
            pltpu.VMEM((_CHUNK * _D,), jnp.float32),
            pltpu.VMEM((_CHUNK * 16,), jnp.float32),
            pltpu.VMEM((_CHUNK * 16,), jnp.float32),
            pltpu.VMEM((_CHUNK,), jnp.int32),
            pltpu.VMEM((_CHUNK,), jnp.int32),
            pltpu.VMEM((_ACCR * _D,), jnp.float32),
            pltpu.VMEM((_ACCR * 16,), jnp.float32),
            pltpu.VMEM((_NW,), jnp.int32),
            pltpu.VMEM((_NW,), jnp.int32),
            pltpu.VMEM((_NW,), jnp.int32),
            pltpu.SemaphoreType.DMA,
            pltpu.SemaphoreType.DMA,
        ],
    )
    def scatter_kernel(big_hbm, s16_hbm, idx_hbm, znum_hbm, zden_hbm,
                       bs_hbm, be_hbm, bn_hbm, num_out, den_out,
                       rows_v0, rows_v1, s16_v0, s16_v1, idx_v0, idx_v1,
                       acc_v, accd_v,
                       bs_v, be_v, bn_v, sem0, sem1):
        c = lax.axis_index("c")
        t = lax.axis_index("s")
        w = t * _NC + c
        lo = w * _SPT

        pltpu.sync_copy(bs_hbm, bs_v)
        pltpu.sync_copy(be_hbm, be_v)
        pltpu.sync_copy(bn_hbm, bn_v)
        g = w // 16
        lane = w - g * 16
        gds = pl.multiple_of(g * 16, 16)
        astart = pl.multiple_of(_extract16(bs_v[pl.ds(gds, 16)], lane), 8)
        aend = _extract16(be_v[pl.ds(gds, 16)], lane)
        nch = _extract16(bn_v[pl.ds(gds, 16)], lane)

        pltpu.sync_copy(znum_hbm, acc_v)
        pltpu.sync_copy(zden_hbm, accd_v)

        def _off(k):
            win = astart + k * _CHUNK
            return win, pl.multiple_of(jnp.minimum(win, n - _CHUNK), 8)

        def _issue(k, rows_b, s16_b, idx_b, sem):
            _, off = _off(k)
            pltpu.async_copy(
                big_hbm.at[pl.ds(pl.multiple_of(off * _D, 8), _CHUNK * _D)],
                rows_b, sem)
            pltpu.async_copy(
                s16_hbm.at[pl.ds(pl.multiple_of(off * 16, 8), _CHUNK * 16)],
                s16_b, sem)
            pltpu.async_copy(idx_hbm.at[pl.ds(off, _CHUNK)], idx_b, sem)

        def _wait(rows_b, s16_b, idx_b, sem):
            pltpu.make_async_copy(big_hbm.at[pl.ds(0, _CHUNK * _D)],
                                  rows_b, sem).wait()
            pltpu.make_async_copy(s16_hbm.at[pl.ds(0, _CHUNK * 16)],
                                  s16_b, sem).wait()
            pltpu.make_async_copy(idx_hbm.at[pl.ds(0, _CHUNK)],
                                  idx_b, sem).wait()

        def _process(k, rows_b, s16_b, idx_b):
            win, off = _off(k)
            for g2 in range(_CHUNK // 16):
                v = idx_b[pl.ds(g2 * 16, 16)]
                rowpos = off + g2 * 16 + lax.iota(jnp.int32, 16)
                local = v - lo
                ok = ((rowpos >= win) & (rowpos < aend)
                      & (local >= 0) & (local < _SPT))
                offs = jnp.where(ok, local, _SPT) * _D
                for l in range(16):
                    r = g2 * 16 + l
                    o = pl.multiple_of(offs[l], 16)
                    od = pl.multiple_of(
                        lax.shift_right_logical(o, 3), 16)
                    plsc.addupdate(accd_v.at[pl.ds(od, 16)],
                                   s16_b[pl.ds(r * 16, 16)])
                    for j in range(_D // 16):
                        plsc.addupdate(
                            acc_v.at[pl.ds(pl.multiple_of(o + j * 16, 16), 16)],
                            rows_b[pl.ds(r * _D + j * 16, 16)])

        @pl.when(nch > 0)
        def _():
            _issue(0, rows_v0, s16_v0, idx_v0, sem0)

        def pair_body(i, carry):
            m1 = 2 * i + 1
            _wait(rows_v0, s16_v0, idx_v0, sem0)

            @pl.when(m1 < nch)
            def _():
                _issue(m1, rows_v1, s16_v1, idx_v1, sem1)

            _process(2 * i, rows_v0, s16_v0, idx_v0)

            @pl.when(m1 < nch)
            def _():
                _wait(rows_v1, s16_v1, idx_v1, sem1)

                @pl.when(m1 + 1 < nch)
                def _():
                    _issue(m1 + 1, rows_v0, s16_v0, idx_v0, sem0)

                _process(m1, rows_v1, s16_v1, idx_v1)

            return carry

        lax.fori_loop(0, (nch + 1) // 2, pair_body, 0)

        pltpu.sync_copy(
            acc_v.at[pl.ds(0, _SPT * _D)],
            num_out.at[pl.ds(pl.multiple_of(lo * _D, 8), _SPT * _D)])
        pltpu.sync_copy(
            accd_v.at[pl.ds(0, _SPT * 16)],
            den_out.at[pl.ds(pl.multiple_of(lo * 16, 8), _SPT * 16)])

    return scatter_kernel


def _div_body(num_ref, den_ref, out_ref):
    out_ref[...] = num_ref[...] / (den_ref[:, 0:1] + _EPS)


def _div_call(num, den):
    bs = min(1000, _S)
    return pl.pallas_call(
        _div_body,
        grid=(_S // bs,),
        in_specs=[
            pl.BlockSpec((bs, _D), lambda i: (i, 0)),
            pl.BlockSpec((bs, 16), lambda i: (i, 0)),
        ],
        out_specs=pl.BlockSpec((bs, _D), lambda i: (i, 0)),
        out_shape=jax.ShapeDtypeStruct((_S, _D), jnp.float32),
    )(num, den)


def kernel(x, index, weights, gW1, gb1, gW2, gb2, mW1, mb1, mW2, mb2, p):
    n = x.shape[0]
    out_big, s16 = _mlp_call(
        x, weights.reshape(-1, 1),
        gW1, gb1.reshape(1, -1), gW2, gb2.reshape(1, 1),
        mW1, mb1.reshape(1, -1), mW2, mb2.reshape(1, -1),
        p.reshape(1, 1))

    cuts = jnp.arange(_NW + 1, dtype=jnp.int32) * _SPT
    b = jnp.searchsorted(index, cuts).astype(jnp.int32)
    astart = (b[:_NW] // 8) * 8
    aend = b[1:]
    nch = (aend - astart + (_CHUNK - 1)) // _CHUNK

    znum = jnp.zeros((_ACCR * _D,), jnp.float32)
    zden = jnp.zeros((_ACCR * 16,), jnp.float32)
    num_f, den_f = _make_scatter_kernel(n)(
        out_big.reshape(-1), s16.reshape(-1), index,
        znum, zden, astart, aend, nch)
    num = num_f.reshape(_SP, _D)[:_S]
    den = den_f.reshape(_SP, 16)[:_S]
    return _div_call(num, den)


# speedup vs baseline: 4.4387x; 1.0994x over previous
import dataclasses
import functools

import jax
import jax.numpy as jnp
from jax import lax
from jax.experimental import pallas as pl
from jax.experimental.pallas import tpu as pltpu
from jax.experimental.pallas import tpu_sc as plsc

_S = 10000
_SP = 10240
_D = 128
_B = 1280
_NC = 2
_NT = 16
_NW = _NC * _NT
_SPT = _SP // _NW
_ACCR = _SPT + 8
_CHUNK = 128
_EPS = 1e-10

_SELU_ALPHA = 1.6732632423543772848170429916717
_SELU_SCALE = 1.0507009873554804934193349852946


def _selu(x):
    return _SELU_SCALE * jnp.where(x > 0, x, _SELU_ALPHA * (jnp.exp(x) - 1.0))


def _mlp_body(x_ref, w_ref, gW1_ref, gb1_ref, gW2_ref, gb2_ref,
              mW1_ref, mb1_ref, mW2_ref, mb2_ref, p_ref,
              out_ref, s_ref):
    x = x_ref[...]
    h = jnp.dot(x, gW1_ref[...], preferred_element_type=jnp.float32)
    h = _selu(h + gb1_ref[...])
    g = jnp.dot(h, gW2_ref[...], preferred_element_type=jnp.float32)
    g = g + gb2_ref[...]
    w = w_ref[...]
    t = p_ref[0, 0] * jnp.log(jnp.maximum(w, _EPS)) + g
    s = jnp.exp(jnp.clip(t, -80.0, 80.0))
    m = _selu(jnp.dot(x, mW1_ref[...], preferred_element_type=jnp.float32)
              + mb1_ref[...])
    m = _selu(jnp.dot(m, mW2_ref[...], preferred_element_type=jnp.float32)
              + mb2_ref[...])
    out_ref[...] = s * m
    s_ref[...] = jnp.concatenate(
        [s, jnp.zeros((s.shape[0], 15), jnp.float32)], axis=1)


def _mlp_call(x, w2d, gW1, gb1, gW2, gb2, mW1, mb1, mW2, mb2, p2d):
    n = x.shape[0]
    grid = (n // _B,)
    full = lambda shape: pl.BlockSpec(shape, lambda i: (0, 0))
    return pl.pallas_call(
        _mlp_body,
        grid=grid,
        in_specs=[
            pl.BlockSpec((_B, _D), lambda i: (i, 0)),
            pl.BlockSpec((_B, 1), lambda i: (i, 0)),
            full((_D, 64)), full((1, 64)),
            full((64, 1)), full((1, 1)),
            full((_D, _D)), full((1, _D)),
            full((_D, _D)), full((1, _D)),
            full((1, 1)),
        ],
        out_specs=[
            pl.BlockSpec((_B, _D), lambda i: (i, 0)),
            pl.BlockSpec((_B, 16), lambda i: (i, 0)),
        ],
        out_shape=[
            jax.ShapeDtypeStruct((n, _D), jnp.float32),
            jax.ShapeDtypeStruct((n, 16), jnp.float32),
        ],
    )(x, w2d, gW1, gb1, gW2, gb2, mW1, mb1, mW2, mb2, p2d)


def _extract16(vec, lane):
    m = lax.iota(jnp.int32, 16) == lane
    return jnp.max(jnp.where(m, vec, 0))


def _make_scatter_kernel(n):
    mesh = plsc.VectorSubcoreMesh(core_axis_name="c", subcore_axis_name="s")
    cp = pltpu.CompilerParams()
    if "needs_layout_passes" in pltpu.CompilerParams.__dataclass_fields__:
        cp = dataclasses.replace(cp, needs_layout_passes=False)

    @functools.partial(
        pl.kernel,
        out_type=(jax.ShapeDtypeStruct((_SP * _D,), jnp.float32),
                  jax.ShapeDtypeStruct((_SP * 16,), jnp.float32)),
        mesh=mesh,
        compiler_params=cp,
        scratch_types=[
            pltpu.VMEM((_CHUNK, _D), jnp.float32),
            pltpu.VMEM((_CHUNK, _D), jnp.float32),
            pltpu.VMEM((_CHUNK, 16), jnp.float32),
            pltpu.VMEM((_CHUNK, 16), jnp.float32),
            pltpu.VMEM((_CHUNK,), jnp.int32),
            pltpu.VMEM((_CHUNK,), jnp.int32),
            pltpu.VMEM((_ACCR * _D,), jnp.float32),
            pltpu.VMEM((_ACCR * 16,), jnp.float32),
            pltpu.VMEM((_NW,), jnp.int32),
            pltpu.VMEM((_NW,), jnp.int32),
            pltpu.VMEM((_NW,), jnp.int32),
            pltpu.SemaphoreType.DMA,
            pltpu.SemaphoreType.DMA,
        ],
    )
    def scatter_kernel(big_hbm, s16_hbm, idx_hbm, znum_hbm, zden_hbm,
                       bs_hbm, be_hbm, bn_hbm, num_out, den_out,
                       rows_v0, rows_v1, s16_v0, s16_v1, idx_v0, idx_v1,
                       acc_v, accd_v,
                       bs_v, be_v, bn_v, sem0, sem1):
        c = lax.axis_index("c")
        t = lax.axis_index("s")
        w = t * _NC + c
        lo = w * _SPT

        pltpu.sync_copy(bs_hbm, bs_v)
        pltpu.sync_copy(be_hbm, be_v)
        pltpu.sync_copy(bn_hbm, bn_v)
        g = w // 16
        lane = w - g * 16
        gds = pl.multiple_of(g * 16, 16)
        astart = pl.multiple_of(_extract16(bs_v[pl.ds(gds, 16)], lane), 8)
        aend = _extract16(be_v[pl.ds(gds, 16)], lane)
        nch = _extract16(bn_v[pl.ds(gds, 16)], lane)

        pltpu.sync_copy(znum_hbm, acc_v)
        pltpu.sync_copy(zden_hbm, accd_v)

        def _off(k):
            win = astart + k * _CHUNK
            return win, pl.multiple_of(jnp.minimum(win, n - _CHUNK), 8)

        def _issue(k, rows_b, s16_b, idx_b, sem):
            _, off = _off(k)
            pltpu.async_copy(big_hbm.at[pl.ds(off, _CHUNK)], rows_b, sem)
            pltpu.async_copy(s16_hbm.at[pl.ds(off, _CHUNK)], s16_b, sem)
            pltpu.async_copy(idx_hbm.at[pl.ds(off, _CHUNK)], idx_b, sem)

        def _wait(rows_b, s16_b, idx_b, sem):
            pltpu.make_async_copy(big_hbm.at[pl.ds(0, _CHUNK)],
                                  rows_b, sem).wait()
            pltpu.make_async_copy(s16_hbm.at[pl.ds(0, _CHUNK)],
                                  s16_b, sem).wait()
            pltpu.make_async_copy(idx_hbm.at[pl.ds(0, _CHUNK)],
                                  idx_b, sem).wait()

        def _process(k, rows_b, s16_b, idx_b):
            win, off = _off(k)
            for g2 in range(_CHUNK // 16):
                v = idx_b[pl.ds(g2 * 16, 16)]
                rowpos = off + g2 * 16 + lax.iota(jnp.int32, 16)
                local = v - lo
                ok = ((rowpos >= win) & (rowpos < aend)
                      & (local >= 0) & (local < _SPT))
                offs = jnp.where(ok, local, _SPT) * _D
                for l in range(16):
                    r = g2 * 16 + l
                    o = pl.multiple_of(offs[l], 16)
                    od = pl.multiple_of(
                        lax.shift_right_logical(o, 3), 16)
                    plsc.addupdate(accd_v.at[pl.ds(od, 16)],
                                   s16_b[r, pl.ds(0, 16)])
                    for j in range(_D // 16):
                        plsc.addupdate(
                            acc_v.at[pl.ds(pl.multiple_of(o + j * 16, 16), 16)],
                            rows_b[r, pl.ds(j * 16, 16)])

        @pl.when(nch > 0)
        def _():
            _issue(0, rows_v0, s16_v0, idx_v0, sem0)

        def pair_body(i, carry):
            m1 = 2 * i + 1
            _wait(rows_v0, s16_v0, idx_v0, sem0)

            @pl.when(m1 < nch)
            def _():
                _issue(m1, rows_v1, s16_v1, idx_v1, sem1)

            _process(2 * i, rows_v0, s16_v0, idx_v0)

            @pl.when(m1 < nch)
            def _():
                _wait(rows_v1, s16_v1, idx_v1, sem1)

                @pl.when(m1 + 1 < nch)
                def _():
                    _issue(m1 + 1, rows_v0, s16_v0, idx_v0, sem0)

                _process(m1, rows_v1, s16_v1, idx_v1)

            return carry

        lax.fori_loop(0, (nch + 1) // 2, pair_body, 0)

        pltpu.sync_copy(
            acc_v.at[pl.ds(0, _SPT * _D)],
            num_out.at[pl.ds(pl.multiple_of(lo * _D, 8), _SPT * _D)])
        pltpu.sync_copy(
            accd_v.at[pl.ds(0, _SPT * 16)],
            den_out.at[pl.ds(pl.multiple_of(lo * 16, 8), _SPT * 16)])

    return scatter_kernel


def _div_body(num_ref, den_ref, out_ref):
    out_ref[...] = num_ref[...] / (den_ref[:, 0:1] + _EPS)


def _div_call(num, den):
    bs = min(1000, _S)
    return pl.pallas_call(
        _div_body,
        grid=(_S // bs,),
        in_specs=[
            pl.BlockSpec((bs, _D), lambda i: (i, 0)),
            pl.BlockSpec((bs, 16), lambda i: (i, 0)),
        ],
        out_specs=pl.BlockSpec((bs, _D), lambda i: (i, 0)),
        out_shape=jax.ShapeDtypeStruct((_S, _D), jnp.float32),
    )(num, den)


def kernel(x, index, weights, gW1, gb1, gW2, gb2, mW1, mb1, mW2, mb2, p):
    n = x.shape[0]
    out_big, s16 = _mlp_call(
        x, weights.reshape(-1, 1),
        gW1, gb1.reshape(1, -1), gW2, gb2.reshape(1, 1),
        mW1, mb1.reshape(1, -1), mW2, mb2.reshape(1, -1),
        p.reshape(1, 1))

    cuts = jnp.arange(_NW + 1, dtype=jnp.int32) * _SPT
    b = jnp.searchsorted(index, cuts).astype(jnp.int32)
    astart = (b[:_NW] // 8) * 8
    aend = b[1:]
    nch = (aend - astart + (_CHUNK - 1)) // _CHUNK

    znum = jnp.zeros((_ACCR * _D,), jnp.float32)
    zden = jnp.zeros((_ACCR * 16,), jnp.float32)
    num_f, den_f = _make_scatter_kernel(n)(
        out_big, s16, index, znum, zden, astart, aend, nch)
    num = num_f.reshape(_SP, _D)[:_S]
    den = den_f.reshape(_SP, 16)[:_S]
    return _div_call(num, den)
